# baseline (speedup 1.0000x reference)
import functools

import jax
import jax.numpy as jnp
from jax import lax
from jax.experimental import pallas as pl
from jax.experimental.pallas import tpu as pltpu
from jax.experimental.pallas import tpu_sc as plsc

N_TOK = 4096 * 50
D = 64
IMG = 128
PK = 128

NC, NS = 2, 16
NW = NC * NS
TOK_PER_W = N_TOK // NW
IDX_VEC = 128
VECS_PER_W = TOK_PER_W // IDX_VEC
CHUNK_VECS = 5
CHUNK_ROWS = CHUNK_VECS * IDX_VEC
N_CHUNKS = VECS_PER_W // CHUNK_VECS


@functools.partial(
    pl.kernel,
    mesh=plsc.VectorSubcoreMesh(core_axis_name="c", subcore_axis_name="s"),
    compiler_params=pltpu.CompilerParams(use_tc_tiling_on_sc=False),
    out_type=jax.ShapeDtypeStruct((N_TOK, PK), jnp.float32),
    scratch_types=[
        pltpu.VMEM((VECS_PER_W, IDX_VEC), jnp.int32),
        pltpu.VMEM((CHUNK_ROWS, PK), jnp.float32),
        pltpu.SemaphoreType.DMA,
    ],
)
def _sc_gather(idx_hbm, table_hbm, out_hbm, idx_v, rows_v, sem):
    wid = lax.axis_index("s") * NC + lax.axis_index("c")
    pltpu.sync_copy(idx_hbm.at[pl.ds(wid * VECS_PER_W, VECS_PER_W)], idx_v)

    def chunk_body(c, carry):
        copies = []
        for j in range(CHUNK_VECS):
            copies.append(
                pltpu.async_copy(
                    table_hbm.at[idx_v.at[c * CHUNK_VECS + j]],
                    rows_v.at[pl.ds(j * IDX_VEC, IDX_VEC)],
                    sem,
                )
            )
        for cp in copies:
            cp.wait()
        base = wid * TOK_PER_W + c * CHUNK_ROWS
        pltpu.sync_copy(rows_v, out_hbm.at[pl.ds(base, CHUNK_ROWS)])
        return carry

    lax.fori_loop(0, N_CHUNKS, chunk_body, 0)


BLK = 1024
N_BLK = N_TOK // BLK


def _tc_blend_body(img_ref, e2_ref, mask_ref, par_ref, w_ref, b_ref, out_ref):
    img = img_ref[...]
    e2 = e2_ref[...]
    m = mask_ref[...]
    p = par_ref[...]
    left = e2[:, :D]
    right = e2[:, D:]
    emb = left + p * (right - left)
    proj = jnp.dot(img, w_ref[...], preferred_element_type=jnp.float32)
    proj = proj + b_ref[...]
    out_ref[...] = emb + m * (proj - emb)


_tc_blend = pl.pallas_call(
    _tc_blend_body,
    grid=(N_BLK,),
    in_specs=[
        pl.BlockSpec((BLK, IMG), lambda i: (i, 0)),
        pl.BlockSpec((BLK, PK), lambda i: (i, 0)),
        pl.BlockSpec((BLK, 1), lambda i: (i, 0)),
        pl.BlockSpec((BLK, 1), lambda i: (i, 0)),
        pl.BlockSpec((IMG, D), lambda i: (0, 0)),
        pl.BlockSpec((1, D), lambda i: (0, 0)),
    ],
    out_specs=pl.BlockSpec((BLK, D), lambda i: (i, 0)),
    out_shape=jax.ShapeDtypeStruct((N_TOK, D), jnp.float32),
)


def kernel(text_input_sequence, image_input_sequence, image_sequence_mask, table, W, b):
    B, L = text_input_sequence.shape
    idx = text_input_sequence.astype(jnp.int32).reshape(N_TOK)
    idx_half = (idx >> 1).reshape(N_TOK // IDX_VEC, IDX_VEC)
    par = (idx & 1).astype(jnp.float32).reshape(N_TOK, 1)
    tablep = table.reshape(table.shape[0] // 2, PK)
    embs2 = _sc_gather(idx_half, tablep)
    img2d = image_input_sequence.reshape(N_TOK, IMG)
    mask2d = image_sequence_mask.astype(jnp.float32).reshape(N_TOK, 1)
    out2d = _tc_blend(img2d, embs2, mask2d, par, W, b.reshape(1, D))
    return out2d.reshape(B, L, D)
